# Initial kernel scaffold; baseline (speedup 1.0000x reference)
import jax
import jax.numpy as jnp
from jax.experimental import pallas as pl


def kernel(feature, graph, W0, W1, bias):
    raise NotImplementedError("write your pallas kernel here")


# speedup vs baseline: 24.0089x; 24.0089x over previous
import functools

import jax
import jax.numpy as jnp
from jax import lax
from jax.experimental import pallas as pl
from jax.experimental.pallas import tpu as pltpu
from jax.experimental.pallas import tpu_sc as plsc

B, N, D, L = 16, 128, 256, 512
_NC, _NS = 2, 16
_NW = _NC * _NS
_ROWS = B * N
_RPW = _ROWS // _NW


def _hist_body(graph_hbm, counts_hbm, g_v, c_v):
    wid = lax.axis_index("s") * _NC + lax.axis_index("c")
    pltpu.sync_copy(graph_hbm.at[pl.ds(wid * (_RPW * N), _RPW * N)], g_v)
    zeros = jnp.zeros((16,), jnp.float32)

    def zero_chunk(k, carry):
        c_v[pl.ds(k * 16, 16)] = zeros
        return carry

    lax.fori_loop(0, _RPW * L // 16, zero_chunk, 0)
    ones = jnp.ones((16,), jnp.float32)

    def scat_row(r, carry):
        row_off = jnp.full((16,), r * L, jnp.int32)
        for k in range(N // 16):
            labels = g_v[pl.ds(r * N + k * 16, 16)]
            plsc.addupdate_scatter(c_v, [row_off + labels], ones)
        return carry

    lax.fori_loop(0, _RPW, scat_row, 0)
    pltpu.sync_copy(c_v, counts_hbm.at[pl.ds(wid * (_RPW * L), _RPW * L)])


_hist = functools.partial(
    pl.kernel,
    mesh=plsc.VectorSubcoreMesh(core_axis_name="c", subcore_axis_name="s"),
    out_type=jax.ShapeDtypeStruct((_ROWS * L,), jnp.float32),
    scratch_types=[
        pltpu.VMEM((_RPW * N,), jnp.int32),
        pltpu.VMEM((_RPW * L,), jnp.float32),
    ],
    compiler_params=pltpu.CompilerParams(needs_layout_passes=False),
)(_hist_body)


def _tc_body(f_ref, w0_ref, w1_ref, bias_ref, c_ref, o_ref):
    f = f_ref[...]
    w = w0_ref[...] + w1_ref[...]
    o = f + lax.dot_general(
        f, w, (((1,), (1,)), ((), ())), preferred_element_type=jnp.float32
    )
    o_ref[...] = o + jnp.dot(
        c_ref[...], bias_ref[...], preferred_element_type=jnp.float32
    )


def kernel(feature, graph, W0, W1, bias):
    g2 = graph.reshape(_ROWS * N).astype(jnp.int32)
    counts = _hist(g2).reshape(_ROWS, L)
    f2 = feature.reshape(_ROWS, D)
    out = pl.pallas_call(
        _tc_body,
        out_shape=jax.ShapeDtypeStruct((_ROWS, D), jnp.float32),
    )(f2, W0, W1, bias, counts)
    return out.reshape(B, N, D)


# speedup vs baseline: 28.6746x; 1.1943x over previous
import functools

import jax
import jax.numpy as jnp
from jax import lax
from jax.experimental import pallas as pl
from jax.experimental.pallas import tpu as pltpu
from jax.experimental.pallas import tpu_sc as plsc

B, N, D, L = 16, 128, 256, 512
_NC, _NS = 2, 16
_NW = _NC * _NS
_ROWS = B * N
_RPW = _ROWS // _NW


def _hist_body(graph_hbm, counts_hbm, g_v, c_v):
    wid = lax.axis_index("s") * _NC + lax.axis_index("c")
    pltpu.sync_copy(graph_hbm.at[pl.ds(wid * (_RPW * N), _RPW * N)], g_v)
    zeros = jnp.zeros((16,), jnp.float32)

    def zero_chunk(k, carry):
        base = k * 256
        for t in range(16):
            c_v[pl.ds(base + t * 16, 16)] = zeros
        return carry

    lax.fori_loop(0, _RPW * L // 256, zero_chunk, 0)
    ones = jnp.ones((16,), jnp.float32)

    def scat_row(r2, carry):
        for h in range(2):
            r = r2 * 2 + h
            row_off = jnp.full((16,), r * L, jnp.int32)
            for k in range(N // 16):
                labels = g_v[pl.ds(r * N + k * 16, 16)]
                plsc.addupdate_scatter(c_v, [row_off + labels], ones)
        return carry

    lax.fori_loop(0, _RPW // 2, scat_row, 0)
    pltpu.sync_copy(c_v, counts_hbm.at[pl.ds(wid * (_RPW * L), _RPW * L)])


_hist = functools.partial(
    pl.kernel,
    mesh=plsc.VectorSubcoreMesh(core_axis_name="c", subcore_axis_name="s"),
    out_type=jax.ShapeDtypeStruct((_ROWS * L,), jnp.float32),
    scratch_types=[
        pltpu.VMEM((_RPW * N,), jnp.int32),
        pltpu.VMEM((_RPW * L,), jnp.float32),
    ],
    compiler_params=pltpu.CompilerParams(needs_layout_passes=False),
)(_hist_body)


def _tc_body(f_ref, w0_ref, w1_ref, bias_ref, c_ref, o_ref):
    f = f_ref[...]
    w = w0_ref[...] + w1_ref[...]
    o = f + lax.dot_general(
        f, w, (((1,), (1,)), ((), ())), preferred_element_type=jnp.float32
    )
    o_ref[...] = o + jnp.dot(
        c_ref[...], bias_ref[...], preferred_element_type=jnp.float32
    )


def kernel(feature, graph, W0, W1, bias):
    g2 = graph.reshape(_ROWS * N).astype(jnp.int32)
    counts = _hist(g2).reshape(_ROWS, L)
    f2 = feature.reshape(_ROWS, D)
    out = pl.pallas_call(
        _tc_body,
        out_shape=jax.ShapeDtypeStruct((_ROWS, D), jnp.float32),
    )(f2, W0, W1, bias, counts)
    return out.reshape(B, N, D)


# speedup vs baseline: 32.0783x; 1.1187x over previous
import functools

import jax
import jax.numpy as jnp
from jax import lax
from jax.experimental import pallas as pl
from jax.experimental.pallas import tpu as pltpu
from jax.experimental.pallas import tpu_sc as plsc

B, N, D, L = 16, 128, 256, 512
_NC, _NS = 2, 16
_NW = _NC * _NS
_ROWS = B * N
_RPW = _ROWS // _NW
_Q = L // 128


def _hist_body(graph_hbm, counts_hbm, g_v, c_v, sem):
    wid = lax.axis_index("s") * _NC + lax.axis_index("c")
    base = wid * _RPW
    cp = pltpu.async_copy(graph_hbm.at[pl.ds(base, _RPW)], g_v, sem)
    zeros = jnp.zeros((16,), jnp.float32)

    def zero_chunk(k, carry):
        for h in range(2):
            for t in range(8):
                c_v[k * 2 + h, pl.ds(t * 16, 16)] = zeros
        return carry

    lax.fori_loop(0, _Q * _RPW // 2, zero_chunk, 0)
    cp.wait()
    ones = jnp.ones((16,), jnp.float32)

    def scat_row(r2, carry):
        for h in range(2):
            lr = r2 * 2 + h
            lr_v = jnp.full((16,), lr, jnp.int32)
            for k in range(N // 16):
                labels = g_v[lr, pl.ds(k * 16, 16)]
                row_idx = lax.shift_right_logical(labels, 7) * _RPW + lr_v
                col_idx = lax.bitwise_and(labels, 127)
                plsc.addupdate_scatter(c_v, [row_idx, col_idx], ones)
        return carry

    lax.fori_loop(0, _RPW // 2, scat_row, 0)
    for q in range(_Q):
        pltpu.sync_copy(c_v.at[pl.ds(q * _RPW, _RPW)],
                        counts_hbm.at[q, pl.ds(base, _RPW)])


_hist = functools.partial(
    pl.kernel,
    mesh=plsc.VectorSubcoreMesh(core_axis_name="c", subcore_axis_name="s"),
    out_type=jax.ShapeDtypeStruct((_Q, _ROWS, 128), jnp.float32),
    scratch_types=[
        pltpu.VMEM((_RPW, N), jnp.int32),
        pltpu.VMEM((_Q * _RPW, 128), jnp.float32),
        pltpu.SemaphoreType.DMA,
    ],
    compiler_params=pltpu.CompilerParams(needs_layout_passes=False),
)(_hist_body)


def _tc_body(f_ref, w0_ref, w1_ref, bias_ref, c_ref, o_ref):
    f = f_ref[...]
    w = w0_ref[...] + w1_ref[...]
    o = f + lax.dot_general(
        f, w, (((1,), (1,)), ((), ())), preferred_element_type=jnp.float32
    )
    for q in range(_Q):
        o = o + jnp.dot(
            c_ref[q], bias_ref[q], preferred_element_type=jnp.float32
        )
    o_ref[...] = o


def kernel(feature, graph, W0, W1, bias):
    g2 = graph.reshape(_ROWS, N).astype(jnp.int32)
    counts = _hist(g2)
    f2 = feature.reshape(_ROWS, D)
    bias4 = bias.reshape(_Q, 128, D)
    out = pl.pallas_call(
        _tc_body,
        out_shape=jax.ShapeDtypeStruct((_ROWS, D), jnp.float32),
    )(f2, W0, W1, bias4, counts)
    return out.reshape(B, N, D)
